# baseline (speedup 1.0000x reference)
import math

import jax
import jax.numpy as jnp
from jax.experimental import pallas as pl


def _attn_kernel(q_ref, k_ref, v_ref, o_ref):
    S = q_ref.shape[2]
    W2 = q_ref.shape[3]
    D = W2 // 2
    L = S.bit_length() - 1
    scale = 1.0 / math.sqrt(D)

    q = q_ref[0, 0, :, :]
    k = k_ref[0, 0, :, :]
    v = v_ref[0, 0, :, :]

    lane = jax.lax.broadcasted_iota(jnp.int32, (W2, 2), 0)
    col = jax.lax.broadcasted_iota(jnp.int32, (W2, 2), 1)
    wred = jnp.where((lane // D) == col, 1.0, 0.0).astype(jnp.float32)
    dnums = (((1,), (0,)), ((), ()))

    def rowdot(a, b):
        return jax.lax.dot_general(a * b, wred, dnums,
                                   preferred_element_type=jnp.float32)

    def lanebcast(e):
        return jnp.broadcast_to(e[:, :, None], (e.shape[0], 2, D)).reshape(
            e.shape[0], W2)

    row = jax.lax.broadcasted_iota(jnp.int32, (S, 1), 0)
    lvl = jax.lax.broadcasted_iota(jnp.int32, (1, 16), 1)
    maskf = ((row >> lvl) & 1).astype(jnp.float32)

    m = rowdot(q, k) * scale
    d = jnp.ones_like(m)
    acc = v

    kl, vl = k, v
    for l in range(L):
        n = S >> l
        kr = kl.reshape(n // 2, 2, W2)
        vr = vl.reshape(n // 2, 2, W2)
        kc0 = kr[:, 0, :]
        kc1 = kr[:, 1, :]
        vc0 = vr[:, 0, :]
        vc1 = vr[:, 1, :]

        rep = 1 << (l + 1)
        nbr_k = jnp.broadcast_to(kc0[:, None, :], (n // 2, rep, W2)).reshape(S, W2)
        nbr_v = jnp.broadcast_to(vc0[:, None, :], (n // 2, rep, W2)).reshape(S, W2)
        s = rowdot(q, nbr_k) * scale
        e = jnp.exp(s - m) * jax.lax.slice(maskf, (0, l), (S, l + 1))
        d = d + e
        acc = acc + lanebcast(e) * nbr_v

        if l + 1 < L:
            kp = 0.5 * (kc0 + kc1)
            s0 = rowdot(kp, kc0) * scale
            s1 = rowdot(kp, kc1) * scale
            ss = 0.5 * (s0 + s1)
            mm = jnp.maximum(jnp.maximum(ss, s0), s1)
            es = jnp.exp(ss - mm)
            e0 = jnp.exp(s0 - mm)
            e1 = jnp.exp(s1 - mm)
            rden = 1.0 / (es + e0 + e1)
            c0 = (0.5 * es + e0) * rden
            c1 = (0.5 * es + e1) * rden
            vl = lanebcast(c0) * vc0 + lanebcast(c1) * vc1
            kl = kp

    o_ref[0, 0, :, :] = acc * lanebcast(1.0 / d)


@jax.jit
def kernel(q, k, v):
    B, S, H, D = q.shape

    def pack(x):
        return x.reshape(B, S, H // 2, 2 * D).transpose(0, 2, 1, 3)

    qt, kt, vt = pack(q), pack(k), pack(v)
    spec = pl.BlockSpec((1, 1, S, 2 * D), lambda b, h: (b, h, 0, 0))
    out = pl.pallas_call(
        _attn_kernel,
        grid=(B, H // 2),
        in_specs=[spec, spec, spec],
        out_specs=spec,
        out_shape=jax.ShapeDtypeStruct((B, H // 2, S, 2 * D), q.dtype),
    )(qt, kt, vt)
    return out.transpose(0, 2, 1, 3).reshape(B, S, H, D)


# speedup vs baseline: 2.5353x; 2.5353x over previous
import math

import jax
import jax.numpy as jnp
from jax.experimental import pallas as pl


def _attn_kernel(q_ref, k_ref, v_ref, o_ref):
    S = q_ref.shape[2]
    W2 = q_ref.shape[3]
    D = W2 // 2
    L = S.bit_length() - 1
    scale = 1.0 / math.sqrt(D)

    q = q_ref[0, 0, :, :]
    k = k_ref[0, 0, :, :]
    v = v_ref[0, 0, :, :]

    lane = jax.lax.broadcasted_iota(jnp.int32, (W2, 2), 0)
    col = jax.lax.broadcasted_iota(jnp.int32, (W2, 2), 1)
    wred = jnp.where((lane // D) == col, 1.0, 0.0).astype(jnp.float32)
    dnums = (((1,), (0,)), ((), ()))

    def rowdot(a, b):
        return jax.lax.dot_general(a * b, wred, dnums,
                                   preferred_element_type=jnp.float32)

    def lanebcast(e):
        n0 = e.shape[0]
        return jnp.concatenate(
            [jnp.broadcast_to(e[:, 0:1], (n0, D)),
             jnp.broadcast_to(e[:, 1:2], (n0, D))], axis=1)

    row = jax.lax.broadcasted_iota(jnp.int32, (S, 1), 0)
    lvl = jax.lax.broadcasted_iota(jnp.int32, (1, 16), 1)
    maskf = ((row >> lvl) & 1).astype(jnp.float32)

    m = rowdot(q, k) * scale
    d = jnp.ones_like(m)
    acc = v

    kl, vl = k, v
    for l in range(L):
        n = S >> l
        kr = kl.reshape(n // 2, 2, W2)
        vr = vl.reshape(n // 2, 2, W2)
        kc0 = kr[:, 0, :]
        kc1 = kr[:, 1, :]
        vc0 = vr[:, 0, :]
        vc1 = vr[:, 1, :]

        rep = 1 << (l + 1)
        nbr_k = jnp.broadcast_to(kc0[:, None, :], (n // 2, rep, W2)).reshape(S, W2)
        nbr_v = jnp.broadcast_to(vc0[:, None, :], (n // 2, rep, W2)).reshape(S, W2)
        s = rowdot(q, nbr_k) * scale
        e = jnp.exp(s - m) * jax.lax.slice(maskf, (0, l), (S, l + 1))
        d = d + e
        acc = acc + lanebcast(e) * nbr_v

        if l + 1 < L:
            kp = 0.5 * (kc0 + kc1)
            s0 = rowdot(kp, kc0) * scale
            s1 = rowdot(kp, kc1) * scale
            ss = 0.5 * (s0 + s1)
            mm = jnp.maximum(jnp.maximum(ss, s0), s1)
            es = jnp.exp(ss - mm)
            e0 = jnp.exp(s0 - mm)
            e1 = jnp.exp(s1 - mm)
            rden = 1.0 / (es + e0 + e1)
            c0 = (0.5 * es + e0) * rden
            c1 = (0.5 * es + e1) * rden
            vl = lanebcast(c0) * vc0 + lanebcast(c1) * vc1
            kl = kp

    o_ref[0, 0, :, :] = acc * lanebcast(1.0 / d)


@jax.jit
def kernel(q, k, v):
    B, S, H, D = q.shape

    def pack(x):
        return x.reshape(B, S, H // 2, 2 * D).transpose(0, 2, 1, 3)

    qt, kt, vt = pack(q), pack(k), pack(v)
    spec = pl.BlockSpec((1, 1, S, 2 * D), lambda b, h: (b, h, 0, 0))
    out = pl.pallas_call(
        _attn_kernel,
        grid=(B, H // 2),
        in_specs=[spec, spec, spec],
        out_specs=spec,
        out_shape=jax.ShapeDtypeStruct((B, H // 2, S, 2 * D), q.dtype),
    )(qt, kt, vt)
    return out.transpose(0, 2, 1, 3).reshape(B, S, H, D)


# speedup vs baseline: 3.6612x; 1.4441x over previous
import math

import jax
import jax.numpy as jnp
from jax.experimental import pallas as pl


def _attn_kernel(q_ref, k_ref, v_ref, o_ref):
    S = q_ref.shape[2]
    W2 = q_ref.shape[3]
    D = W2 // 2
    L = S.bit_length() - 1
    NS = 2 * ((L + 15) // 16) * 16
    scale = 1.0 / math.sqrt(D)
    f32 = jnp.float32
    dnums = (((1,), (0,)), ((), ()))

    def mm(a, b):
        return jax.lax.dot_general(a, b, dnums, preferred_element_type=f32)

    q = q_ref[0, 0, :, :]
    k = k_ref[0, 0, :, :]
    v = v_ref[0, 0, :, :]

    lane_r = jax.lax.broadcasted_iota(jnp.int32, (W2, 2), 0) // D
    col2 = jax.lax.broadcasted_iota(jnp.int32, (W2, 2), 1)
    wred = (lane_r == col2).astype(f32)

    lane_rs = jax.lax.broadcasted_iota(jnp.int32, (W2, NS), 0) // D
    col_s = jax.lax.broadcasted_iota(jnp.int32, (W2, NS), 1)

    def wred_at(l):
        return (col_s == 2 * l + lane_rs).astype(f32)

    row2 = jax.lax.broadcasted_iota(jnp.int32, (2, NS), 0)
    colsn = jax.lax.broadcasted_iota(jnp.int32, (2, NS), 1)
    tsel = (row2 == (colsn % 2)).astype(f32)

    rowsn = jax.lax.broadcasted_iota(jnp.int32, (NS, 2), 0)
    coln2 = jax.lax.broadcasted_iota(jnp.int32, (NS, 2), 1)
    gsum = (coln2 == (rowsn % 2)).astype(f32)
    gsum = gsum * (rowsn < 2 * L).astype(f32)

    rowb = jax.lax.broadcasted_iota(jnp.int32, (2, W2), 0)
    laneb = jax.lax.broadcasted_iota(jnp.int32, (2, W2), 1) // D
    bful = (rowb == laneb).astype(f32)

    rowbs = jax.lax.broadcasted_iota(jnp.int32, (NS, W2), 0)
    lanebs = jax.lax.broadcasted_iota(jnp.int32, (NS, W2), 1) // D

    def bsel_at(l):
        return (rowbs == 2 * l + lanebs).astype(f32)

    row = jax.lax.broadcasted_iota(jnp.int32, (S, 1), 0)
    lvlL = jax.lax.broadcasted_iota(jnp.int32, (1, NS), 1) // 2
    maskf = ((row >> lvlL) & 1).astype(f32)

    m_raw = mm(q * k, wred)

    scores = jnp.zeros((S, NS), f32)
    kl, vl = k, v
    ev_vs = []
    for l in range(L):
        n = S >> l
        kr = kl.reshape(n // 2, 2, W2)
        vr = vl.reshape(n // 2, 2, W2)
        kc0 = kr[:, 0, :]
        kc1 = kr[:, 1, :]
        vc0 = vr[:, 0, :]
        vc1 = vr[:, 1, :]
        ev_vs.append(vc0)

        rep = 1 << (l + 1)
        nbr_k = jnp.broadcast_to(kc0[:, None, :], (n // 2, rep, W2)).reshape(S, W2)
        scores = scores + mm(q * nbr_k, wred_at(l))

        if l + 1 < L:
            kp = 0.5 * (kc0 + kc1)
            s0 = mm(kp * kc0, wred) * scale
            s1 = mm(kp * kc1, wred) * scale
            ss = 0.5 * (s0 + s1)
            mx = jnp.maximum(jnp.maximum(ss, s0), s1)
            es = jnp.exp(ss - mx)
            e0 = jnp.exp(s0 - mx)
            e1 = jnp.exp(s1 - mx)
            rden = 1.0 / (es + e0 + e1)
            c0 = (0.5 * es + e0) * rden
            c1 = (0.5 * es + e1) * rden
            vl = mm(c0, bful) * vc0 + mm(c1, bful) * vc1
            kl = kp

    E = jnp.exp(scale * (scores - mm(m_raw, tsel))) * maskf
    d = 1.0 + mm(E, gsum)

    acc = v
    for l in range(L):
        n2 = (S >> l) // 2
        rep = 1 << (l + 1)
        nbr_v = jnp.broadcast_to(
            ev_vs[l][:, None, :], (n2, rep, W2)).reshape(S, W2)
        acc = acc + mm(E, bsel_at(l)) * nbr_v

    o_ref[0, 0, :, :] = acc * mm(1.0 / d, bful)


@jax.jit
def kernel(q, k, v):
    B, S, H, D = q.shape

    def pack(x):
        return x.reshape(B, S, H // 2, 2 * D).transpose(0, 2, 1, 3)

    qt, kt, vt = pack(q), pack(k), pack(v)
    spec = pl.BlockSpec((1, 1, S, 2 * D), lambda b, h: (b, h, 0, 0))
    out = pl.pallas_call(
        _attn_kernel,
        grid=(B, H // 2),
        in_specs=[spec, spec, spec],
        out_specs=spec,
        out_shape=jax.ShapeDtypeStruct((B, H // 2, S, 2 * D), q.dtype),
    )(qt, kt, vt)
    return out.transpose(0, 2, 1, 3).reshape(B, S, H, D)
